# baseline (speedup 1.0000x reference)
import functools

import jax
import jax.numpy as jnp
from jax import lax
from jax.experimental import pallas as pl
from jax.experimental.pallas import tpu as pltpu
from jax.experimental.pallas import tpu_sc as plsc

B = 16384
XN = 26
H = 64
V = 1_000_000
KT = XN // 2
R2 = KT * B
NC, NS = 2, 16
NW = NC * NS
PW = R2 // NW
CHUNK = 128
CH = PW // CHUNK
TCOLS = V // 128
VFULL = TCOLS * 128
TK = -(-TCOLS // NW)

_MESH = plsc.VectorSubcoreMesh(core_axis_name="c", subcore_axis_name="s")


@functools.partial(
    pl.kernel,
    out_type=jax.ShapeDtypeStruct((V // 2, 2 * H), jnp.float32),
    mesh=_MESH,
    scratch_types=[
        pltpu.VMEM((H, 128), jnp.float32),
        pltpu.VMEM((H, 128), jnp.float32),
        pltpu.VMEM((H, 128), jnp.float32),
        pltpu.SemaphoreType.DMA,
    ],
    compiler_params=pltpu.CompilerParams(needs_layout_passes=False),
)
def _sc_transpose(tt_hbm, tlast_hbm, out_hbm, vin, vout, vlast, sem):
    wid = lax.axis_index("s") * NC + lax.axis_index("c")
    lanes = lax.iota(jnp.int32, 16)

    def col_body(k, carry):
        c = wid + k * NW

        @pl.when(c < TCOLS)
        def _():
            pltpu.async_copy(
                tt_hbm.at[:, pl.ds(c * 128, 128)], vin, sem).wait()

            def row_body(r, carry2):
                for half in range(2):
                    c2 = jnp.full((16,), 2 * r + half, jnp.int32)
                    for t in range(4):
                        vals = plsc.load_gather(vin, [16 * t + lanes, c2])
                        vout[r, pl.ds(half * 64 + t * 16, 16)] = vals
                return carry2

            lax.fori_loop(0, H, row_body, 0)
            pltpu.sync_copy(vout, out_hbm.at[pl.ds(c * H, H)])

        return carry

    lax.fori_loop(0, TK, col_body, 0)

    @pl.when(wid == 0)
    def _():
        pltpu.sync_copy(tlast_hbm, vlast)

        def lrow_body(r, carry2):
            for half in range(2):
                c2 = jnp.full((16,), 2 * r + half, jnp.int32)
                for t in range(4):
                    vals = plsc.load_gather(vlast, [16 * t + lanes, c2])
                    vout[r, pl.ds(half * 64 + t * 16, 16)] = vals
            return carry2

        lax.fori_loop(0, H // 2, lrow_body, 0)
        pltpu.sync_copy(
            vout.at[pl.ds(0, H // 2)],
            out_hbm.at[pl.ds(VFULL // 2, H // 2)])


@functools.partial(
    pl.kernel,
    out_type=jax.ShapeDtypeStruct((R2, 2 * H), jnp.float32),
    mesh=_MESH,
    scratch_types=[
        pltpu.VMEM((CH, CHUNK), jnp.int32),
        pltpu.VMEM((CH, CHUNK), jnp.int32),
        pltpu.VMEM((CHUNK, H), jnp.float32),
        pltpu.VMEM((CHUNK, H), jnp.float32),
        pltpu.SemaphoreType.DMA,
    ],
    compiler_params=pltpu.CompilerParams(use_tc_tiling_on_sc=False),
)
def _sc_gather(idxe_hbm, idxo_hbm, table_hbm, out_hbm,
               idxe_v, idxo_v, bufe, bufo, sem):
    wid = lax.axis_index("s") * NC + lax.axis_index("c")
    base = wid * PW
    pltpu.sync_copy(idxe_hbm.at[wid], idxe_v)
    pltpu.sync_copy(idxo_hbm.at[wid], idxo_v)

    def step(j, carry):
        cpe = pltpu.async_copy(table_hbm.at[idxe_v.at[j]], bufe, sem)
        cpo = pltpu.async_copy(table_hbm.at[idxo_v.at[j]], bufo, sem)
        cpe.wait()
        cpo.wait()
        r0 = base + j * CHUNK
        pltpu.sync_copy(bufe, out_hbm.at[pl.ds(r0, CHUNK), pl.ds(0, H)])
        pltpu.sync_copy(bufo, out_hbm.at[pl.ds(r0, CHUNK), pl.ds(H, H)])
        return carry

    lax.fori_loop(0, CH, step, 0)


def _tc_matmul_body(e_ref, w_ref, b_ref, o_ref):
    t = pl.program_id(1)
    acc = jnp.dot(e_ref[...], w_ref[...], preferred_element_type=jnp.float32)

    @pl.when(t == 0)
    def _():
        o_ref[...] = acc + b_ref[...]

    @pl.when(t != 0)
    def _():
        o_ref[...] = o_ref[...] + acc


def _tc_matmul(emb2, Wt, b2):
    BM = 2048
    return pl.pallas_call(
        _tc_matmul_body,
        grid=(B // BM, KT),
        in_specs=[
            pl.BlockSpec((BM, 2 * H), lambda i, t: (t * (B // BM) + i, 0)),
            pl.BlockSpec((2 * H, H), lambda i, t: (t, 0)),
            pl.BlockSpec((1, H), lambda i, t: (0, 0)),
        ],
        out_specs=pl.BlockSpec((BM, H), lambda i, t: (i, 0)),
        out_shape=jax.ShapeDtypeStruct((B, H), jnp.float32),
    )(emb2, Wt, b2)


def kernel(x, table, W, b):
    tT = table.T
    tl = jnp.pad(tT[:, VFULL:], ((0, 0), (0, 64)))
    tp = _sc_transpose(tT, tl)
    trm = tp.reshape(-1).reshape(V, H)
    xr = x.astype(jnp.int32).reshape(B, KT, 2)
    idxe = xr[:, :, 0].T.reshape(NW, CH, CHUNK)
    idxo = xr[:, :, 1].T.reshape(NW, CH, CHUNK)
    emb2 = _sc_gather(idxe, idxo, trm)
    return _tc_matmul(emb2, W.T, b.reshape(1, H))


# speedup vs baseline: 1.5083x; 1.5083x over previous
import functools

import jax
import jax.numpy as jnp
from jax import lax
from jax.experimental import pallas as pl
from jax.experimental.pallas import tpu as pltpu
from jax.experimental.pallas import tpu_sc as plsc

B = 16384
XN = 26
H = 64
V = 1_000_000
KT = XN // 2
R2 = KT * B
NC, NS = 2, 16
NW = NC * NS
PW = R2 // NW
CHUNK = 128
CH = PW // CHUNK
TCOLS = V // 128
VFULL = TCOLS * 128
TK = -(-TCOLS // NW)

_MESH = plsc.VectorSubcoreMesh(core_axis_name="c", subcore_axis_name="s")


@functools.partial(
    pl.kernel,
    out_type=jax.ShapeDtypeStruct((V // 2, 2 * H), jnp.float32),
    mesh=_MESH,
    scratch_types=[
        pltpu.VMEM((H, 128), jnp.float32),
        pltpu.VMEM((H, 128), jnp.float32),
        pltpu.VMEM((H, 128), jnp.float32),
        pltpu.SemaphoreType.DMA,
    ],
    compiler_params=pltpu.CompilerParams(needs_layout_passes=False),
)
def _sc_transpose(tt_hbm, tlast_hbm, out_hbm, vin, vout, vlast, sem):
    wid = lax.axis_index("s") * NC + lax.axis_index("c")
    lanes = lax.iota(jnp.int32, 16)

    def col_body(k, carry):
        c = wid + k * NW

        @pl.when(c < TCOLS)
        def _():
            pltpu.async_copy(
                tt_hbm.at[:, pl.ds(c * 128, 128)], vin, sem).wait()

            @plsc.parallel_loop(0, H, unroll=8)
            def row_body(r):
                for half in range(2):
                    c2 = jnp.full((16,), 2 * r + half, jnp.int32)
                    for t in range(4):
                        vals = plsc.load_gather(vin, [16 * t + lanes, c2])
                        vout[r, pl.ds(half * 64 + t * 16, 16)] = vals
            pltpu.sync_copy(vout, out_hbm.at[pl.ds(c * H, H)])

        return carry

    lax.fori_loop(0, TK, col_body, 0)

    @pl.when(wid == 0)
    def _():
        pltpu.sync_copy(tlast_hbm, vlast)

        @plsc.parallel_loop(0, H // 2, unroll=8)
        def lrow_body(r):
            for half in range(2):
                c2 = jnp.full((16,), 2 * r + half, jnp.int32)
                for t in range(4):
                    vals = plsc.load_gather(vlast, [16 * t + lanes, c2])
                    vout[r, pl.ds(half * 64 + t * 16, 16)] = vals
        pltpu.sync_copy(
            vout.at[pl.ds(0, H // 2)],
            out_hbm.at[pl.ds(VFULL // 2, H // 2)])


@functools.partial(
    pl.kernel,
    out_type=jax.ShapeDtypeStruct((R2, 2 * H), jnp.float32),
    mesh=_MESH,
    scratch_types=[
        pltpu.VMEM((CH, CHUNK), jnp.int32),
        pltpu.VMEM((CH, CHUNK), jnp.int32),
        pltpu.VMEM((CHUNK, H), jnp.float32),
        pltpu.VMEM((CHUNK, H), jnp.float32),
        pltpu.SemaphoreType.DMA,
    ],
    compiler_params=pltpu.CompilerParams(use_tc_tiling_on_sc=False),
)
def _sc_gather(idxe_hbm, idxo_hbm, table_hbm, out_hbm,
               idxe_v, idxo_v, bufe, bufo, sem):
    wid = lax.axis_index("s") * NC + lax.axis_index("c")
    base = wid * PW
    pltpu.sync_copy(idxe_hbm.at[wid], idxe_v)
    pltpu.sync_copy(idxo_hbm.at[wid], idxo_v)

    def step(j, carry):
        cpe = pltpu.async_copy(table_hbm.at[idxe_v.at[j]], bufe, sem)
        cpo = pltpu.async_copy(table_hbm.at[idxo_v.at[j]], bufo, sem)
        cpe.wait()
        cpo.wait()
        r0 = base + j * CHUNK
        pltpu.sync_copy(bufe, out_hbm.at[pl.ds(r0, CHUNK), pl.ds(0, H)])
        pltpu.sync_copy(bufo, out_hbm.at[pl.ds(r0, CHUNK), pl.ds(H, H)])
        return carry

    lax.fori_loop(0, CH, step, 0)


def _tc_matmul_body(e_ref, w_ref, b_ref, o_ref):
    t = pl.program_id(1)
    acc = jnp.dot(e_ref[...], w_ref[...], preferred_element_type=jnp.float32)

    @pl.when(t == 0)
    def _():
        o_ref[...] = acc + b_ref[...]

    @pl.when(t != 0)
    def _():
        o_ref[...] = o_ref[...] + acc


def _tc_matmul(emb2, Wt, b2):
    BM = 2048
    return pl.pallas_call(
        _tc_matmul_body,
        grid=(B // BM, KT),
        in_specs=[
            pl.BlockSpec((BM, 2 * H), lambda i, t: (t * (B // BM) + i, 0)),
            pl.BlockSpec((2 * H, H), lambda i, t: (t, 0)),
            pl.BlockSpec((1, H), lambda i, t: (0, 0)),
        ],
        out_specs=pl.BlockSpec((BM, H), lambda i, t: (i, 0)),
        out_shape=jax.ShapeDtypeStruct((B, H), jnp.float32),
    )(emb2, Wt, b2)


def kernel(x, table, W, b):
    tT = table.T
    tl = jnp.pad(tT[:, VFULL:], ((0, 0), (0, 64)))
    tp = _sc_transpose(tT, tl)
    trm = tp.reshape(-1).reshape(V, H)
    xr = x.astype(jnp.int32).reshape(B, KT, 2)
    idxe = xr[:, :, 0].T.reshape(NW, CH, CHUNK)
    idxo = xr[:, :, 1].T.reshape(NW, CH, CHUNK)
    emb2 = _sc_gather(idxe, idxo, trm)
    return _tc_matmul(emb2, W.T, b.reshape(1, H))


# speedup vs baseline: 3.0522x; 2.0236x over previous
import functools

import jax
import jax.numpy as jnp
from jax import lax
from jax.experimental import pallas as pl
from jax.experimental.pallas import tpu as pltpu
from jax.experimental.pallas import tpu_sc as plsc

B = 16384
XN = 26
H = 64
V = 1_000_000
KT = XN // 2
R2 = KT * B
NC, NS = 2, 16
NW = NC * NS
PW = R2 // NW
CHUNK = 128
CH = PW // CHUNK
TCOLS = V // 128
VFULL = TCOLS * 128
TK = -(-TCOLS // NW)

_MESH = plsc.VectorSubcoreMesh(core_axis_name="c", subcore_axis_name="s")


BN = 2048
NBLK = -(-V // BN)
VP = NBLK * BN


def _tc_transpose_body(t_ref, o_ref):
    xt = t_ref[...].T
    o_ref[...] = jnp.concatenate([xt[: BN // 2], xt[BN // 2 :]], axis=1)


def _tc_transpose(tT):
    return pl.pallas_call(
        _tc_transpose_body,
        grid=(NBLK,),
        in_specs=[pl.BlockSpec((H, BN), lambda i: (0, i))],
        out_specs=pl.BlockSpec((BN // 2, 2 * H), lambda i: (i, 0)),
        out_shape=jax.ShapeDtypeStruct((VP // 2, 2 * H), jnp.float32),
    )(tT)


@functools.partial(
    pl.kernel,
    out_type=jax.ShapeDtypeStruct((R2, 2 * H), jnp.float32),
    mesh=_MESH,
    scratch_types=[
        pltpu.VMEM((CH, CHUNK), jnp.int32),
        pltpu.VMEM((CH, CHUNK), jnp.int32),
        pltpu.VMEM((CHUNK, H), jnp.float32),
        pltpu.VMEM((CHUNK, H), jnp.float32),
        pltpu.SemaphoreType.DMA,
    ],
    compiler_params=pltpu.CompilerParams(use_tc_tiling_on_sc=False),
)
def _sc_gather(idxe_hbm, idxo_hbm, table_hbm, out_hbm,
               idxe_v, idxo_v, bufe, bufo, sem):
    wid = lax.axis_index("s") * NC + lax.axis_index("c")
    base = wid * PW
    pltpu.sync_copy(idxe_hbm.at[wid], idxe_v)
    pltpu.sync_copy(idxo_hbm.at[wid], idxo_v)

    def step(j, carry):
        cpe = pltpu.async_copy(table_hbm.at[idxe_v.at[j]], bufe, sem)
        cpo = pltpu.async_copy(table_hbm.at[idxo_v.at[j]], bufo, sem)
        cpe.wait()
        cpo.wait()
        r0 = base + j * CHUNK
        pltpu.sync_copy(bufe, out_hbm.at[pl.ds(r0, CHUNK), pl.ds(0, H)])
        pltpu.sync_copy(bufo, out_hbm.at[pl.ds(r0, CHUNK), pl.ds(H, H)])
        return carry

    lax.fori_loop(0, CH, step, 0)


def _tc_matmul_body(e_ref, w_ref, b_ref, o_ref):
    t = pl.program_id(1)
    acc = jnp.dot(e_ref[...], w_ref[...], preferred_element_type=jnp.float32)

    @pl.when(t == 0)
    def _():
        o_ref[...] = acc + b_ref[...]

    @pl.when(t != 0)
    def _():
        o_ref[...] = o_ref[...] + acc


def _tc_matmul(emb2, Wt, b2):
    BM = 2048
    return pl.pallas_call(
        _tc_matmul_body,
        grid=(B // BM, KT),
        in_specs=[
            pl.BlockSpec((BM, 2 * H), lambda i, t: (t * (B // BM) + i, 0)),
            pl.BlockSpec((2 * H, H), lambda i, t: (t, 0)),
            pl.BlockSpec((1, H), lambda i, t: (0, 0)),
        ],
        out_specs=pl.BlockSpec((BM, H), lambda i, t: (i, 0)),
        out_shape=jax.ShapeDtypeStruct((B, H), jnp.float32),
    )(emb2, Wt, b2)


def kernel(x, table, W, b):
    tp = _tc_transpose(table.T)
    trm = tp.reshape(-1).reshape(VP, H)
    xi = x.astype(jnp.int32)
    jm = xi % BN
    xp = (xi - jm) + 2 * (jm % (BN // 2)) + jm // (BN // 2)
    xr = xp.reshape(B, KT, 2)
    idxe = xr[:, :, 0].T.reshape(NW, CH, CHUNK)
    idxo = xr[:, :, 1].T.reshape(NW, CH, CHUNK)
    emb2 = _sc_gather(idxe, idxo, trm)
    return _tc_matmul(emb2, W.T, b.reshape(1, H))


# speedup vs baseline: 4.4999x; 1.4743x over previous
import functools

import jax
import jax.numpy as jnp
from jax import lax
from jax.experimental import pallas as pl
from jax.experimental.pallas import tpu as pltpu
from jax.experimental.pallas import tpu_sc as plsc

B = 16384
XN = 26
H = 64
V = 1_000_000
KT = XN // 2
R2 = KT * B
NC, NS = 2, 16
NW = NC * NS
PW = R2 // NW
CHUNK = 128
CH = PW // CHUNK
TCOLS = V // 128
VFULL = TCOLS * 128
TK = -(-TCOLS // NW)

_MESH = plsc.VectorSubcoreMesh(core_axis_name="c", subcore_axis_name="s")


BN = 16384
NBLK = -(-V // BN)
VP = NBLK * BN


def _tc_transpose_body(t_ref, o_ref):
    xt = t_ref[...].T
    o_ref[...] = jnp.concatenate([xt[: BN // 2], xt[BN // 2 :]], axis=1)


def _tc_transpose(tT):
    return pl.pallas_call(
        _tc_transpose_body,
        grid=(NBLK,),
        in_specs=[pl.BlockSpec((H, BN), lambda i: (0, i))],
        out_specs=pl.BlockSpec((BN // 2, 2 * H), lambda i: (i, 0)),
        out_shape=jax.ShapeDtypeStruct((VP // 2, 2 * H), jnp.float32),
    )(tT)


@functools.partial(
    pl.kernel,
    out_type=jax.ShapeDtypeStruct((R2, 2 * H), jnp.float32),
    mesh=_MESH,
    scratch_types=[
        pltpu.VMEM((CH, CHUNK), jnp.int32),
        pltpu.VMEM((CH, CHUNK), jnp.int32),
        pltpu.VMEM((CHUNK, H), jnp.float32),
        pltpu.VMEM((CHUNK, H), jnp.float32),
        pltpu.SemaphoreType.DMA,
    ],
    compiler_params=pltpu.CompilerParams(use_tc_tiling_on_sc=False),
)
def _sc_gather(idxe_hbm, idxo_hbm, table_hbm, out_hbm,
               idxe_v, idxo_v, bufe, bufo, sem):
    wid = lax.axis_index("s") * NC + lax.axis_index("c")
    base = wid * PW
    pltpu.sync_copy(idxe_hbm.at[wid], idxe_v)
    pltpu.sync_copy(idxo_hbm.at[wid], idxo_v)

    def step(j, carry):
        cpe = pltpu.async_copy(table_hbm.at[idxe_v.at[j]], bufe, sem)
        cpo = pltpu.async_copy(table_hbm.at[idxo_v.at[j]], bufo, sem)
        cpe.wait()
        cpo.wait()
        r0 = base + j * CHUNK
        pltpu.sync_copy(bufe, out_hbm.at[pl.ds(r0, CHUNK), pl.ds(0, H)])
        pltpu.sync_copy(bufo, out_hbm.at[pl.ds(r0, CHUNK), pl.ds(H, H)])
        return carry

    lax.fori_loop(0, CH, step, 0)


def _tc_matmul_body(e_ref, w_ref, b_ref, o_ref):
    t = pl.program_id(1)
    acc = jnp.dot(e_ref[...], w_ref[...], preferred_element_type=jnp.float32)

    @pl.when(t == 0)
    def _():
        o_ref[...] = acc + b_ref[...]

    @pl.when(t != 0)
    def _():
        o_ref[...] = o_ref[...] + acc


def _tc_matmul(emb2, Wt, b2):
    BM = 2048
    return pl.pallas_call(
        _tc_matmul_body,
        grid=(B // BM, KT),
        in_specs=[
            pl.BlockSpec((BM, 2 * H), lambda i, t: (t * (B // BM) + i, 0)),
            pl.BlockSpec((2 * H, H), lambda i, t: (t, 0)),
            pl.BlockSpec((1, H), lambda i, t: (0, 0)),
        ],
        out_specs=pl.BlockSpec((BM, H), lambda i, t: (i, 0)),
        out_shape=jax.ShapeDtypeStruct((B, H), jnp.float32),
    )(emb2, Wt, b2)


def kernel(x, table, W, b):
    tp = _tc_transpose(table.T)
    trm = tp.reshape(-1).reshape(VP, H)
    xi = x.astype(jnp.int32)
    jm = xi % BN
    xp = (xi - jm) + 2 * (jm % (BN // 2)) + jm // (BN // 2)
    xr = xp.reshape(B, KT, 2)
    idxe = xr[:, :, 0].T.reshape(NW, CH, CHUNK)
    idxo = xr[:, :, 1].T.reshape(NW, CH, CHUNK)
    emb2 = _sc_gather(idxe, idxo, trm)
    return _tc_matmul(emb2, W.T, b.reshape(1, H))


# speedup vs baseline: 4.6548x; 1.0344x over previous
import functools

import jax
import jax.numpy as jnp
from jax import lax
from jax.experimental import pallas as pl
from jax.experimental.pallas import tpu as pltpu
from jax.experimental.pallas import tpu_sc as plsc

B = 16384
XN = 26
H = 64
V = 1_000_000
KT = XN // 2
R2 = KT * B
NC, NS = 2, 16
NW = NC * NS
PW = R2 // NW
CHUNK = 128
CH = PW // CHUNK
TCOLS = V // 128
VFULL = TCOLS * 128
TK = -(-TCOLS // NW)

_MESH = plsc.VectorSubcoreMesh(core_axis_name="c", subcore_axis_name="s")


BN = 32768
NBLK = -(-V // BN)
VP = NBLK * BN


def _tc_transpose_body(t_ref, o_ref):
    xt = t_ref[...].T
    o_ref[...] = jnp.concatenate([xt[: BN // 2], xt[BN // 2 :]], axis=1)


def _tc_transpose(tT):
    return pl.pallas_call(
        _tc_transpose_body,
        grid=(NBLK,),
        in_specs=[pl.BlockSpec((H, BN), lambda i: (0, i))],
        out_specs=pl.BlockSpec((BN // 2, 2 * H), lambda i: (i, 0)),
        out_shape=jax.ShapeDtypeStruct((VP // 2, 2 * H), jnp.float32),
    )(tT)


@functools.partial(
    pl.kernel,
    out_type=jax.ShapeDtypeStruct((R2, 2 * H), jnp.float32),
    mesh=_MESH,
    scratch_types=[
        pltpu.VMEM((CH, CHUNK), jnp.int32),
        pltpu.VMEM((CH, CHUNK), jnp.int32),
        pltpu.VMEM((CHUNK, H), jnp.float32),
        pltpu.VMEM((CHUNK, H), jnp.float32),
        pltpu.SemaphoreType.DMA,
    ],
    compiler_params=pltpu.CompilerParams(use_tc_tiling_on_sc=False),
)
def _sc_gather(idxe_hbm, idxo_hbm, table_hbm, out_hbm,
               idxe_v, idxo_v, bufe, bufo, sem):
    wid = lax.axis_index("s") * NC + lax.axis_index("c")
    base = wid * PW
    pltpu.sync_copy(idxe_hbm.at[wid], idxe_v)
    pltpu.sync_copy(idxo_hbm.at[wid], idxo_v)

    def step(j, carry):
        cpe = pltpu.async_copy(table_hbm.at[idxe_v.at[j]], bufe, sem)
        cpo = pltpu.async_copy(table_hbm.at[idxo_v.at[j]], bufo, sem)
        cpe.wait()
        cpo.wait()
        r0 = base + j * CHUNK
        pltpu.sync_copy(bufe, out_hbm.at[pl.ds(r0, CHUNK), pl.ds(0, H)])
        pltpu.sync_copy(bufo, out_hbm.at[pl.ds(r0, CHUNK), pl.ds(H, H)])
        return carry

    lax.fori_loop(0, CH, step, 0)


def _tc_matmul_body(e_ref, w_ref, b_ref, o_ref):
    t = pl.program_id(1)
    acc = jnp.dot(e_ref[...], w_ref[...], preferred_element_type=jnp.float32)

    @pl.when(t == 0)
    def _():
        o_ref[...] = acc + b_ref[...]

    @pl.when(t != 0)
    def _():
        o_ref[...] = o_ref[...] + acc


def _tc_matmul(emb2, Wt, b2):
    BM = 2048
    return pl.pallas_call(
        _tc_matmul_body,
        grid=(B // BM, KT),
        in_specs=[
            pl.BlockSpec((BM, 2 * H), lambda i, t: (t * (B // BM) + i, 0)),
            pl.BlockSpec((2 * H, H), lambda i, t: (t, 0)),
            pl.BlockSpec((1, H), lambda i, t: (0, 0)),
        ],
        out_specs=pl.BlockSpec((BM, H), lambda i, t: (i, 0)),
        out_shape=jax.ShapeDtypeStruct((B, H), jnp.float32),
    )(emb2, Wt, b2)


def kernel(x, table, W, b):
    tp = _tc_transpose(table.T)
    trm = tp.reshape(-1).reshape(VP, H)
    xi = x.astype(jnp.int32)
    jm = xi % BN
    xp = (xi - jm) + 2 * (jm % (BN // 2)) + jm // (BN // 2)
    xr = xp.reshape(B, KT, 2)
    idxe = xr[:, :, 0].T.reshape(NW, CH, CHUNK)
    idxo = xr[:, :, 1].T.reshape(NW, CH, CHUNK)
    emb2 = _sc_gather(idxe, idxo, trm)
    return _tc_matmul(emb2, W.T, b.reshape(1, H))
